# baseline (speedup 1.0000x reference)
import jax
import jax.numpy as jnp
from jax.experimental import pallas as pl
from jax.experimental.pallas import tpu as pltpu

M = 65536
B = 256
NCHUNK = 64
CH = M // NCHUNK
SLOTS = 8
LOOKAHEAD = 6


def _copy_body(i0_ref, i1_ref, o0_ref, o1_ref, buf, load_sems, store_sems):
    srcs = (i0_ref, i1_ref)
    dsts = (o0_ref, o1_ref)
    ntask = 2 * NCHUNK

    def load(t):
        a, c = t % 2, t // 2
        s = t % SLOTS
        pltpu.make_async_copy(
            srcs[a].at[pl.ds(c * CH, CH), :], buf.at[s], load_sems.at[s]
        ).start()

    def store(t):
        a, c = t % 2, t // 2
        s = t % SLOTS
        pltpu.make_async_copy(
            buf.at[s], dsts[a].at[pl.ds(c * CH, CH), :], store_sems.at[s]
        ).start()

    def wait_load(t):
        a, c = t % 2, t // 2
        s = t % SLOTS
        pltpu.make_async_copy(
            srcs[a].at[pl.ds(c * CH, CH), :], buf.at[s], load_sems.at[s]
        ).wait()

    def wait_store(t):
        a, c = t % 2, t // 2
        s = t % SLOTS
        pltpu.make_async_copy(
            buf.at[s], dsts[a].at[pl.ds(c * CH, CH), :], store_sems.at[s]
        ).wait()

    for t in range(LOOKAHEAD):
        load(t)
    for t in range(ntask):
        wait_load(t)
        store(t)
        u = t + LOOKAHEAD
        if u < ntask:
            if u >= SLOTS:
                wait_store(u - SLOTS)
            load(u)
    for t in range(ntask - SLOTS, ntask):
        wait_store(t)


def kernel(input0, input1, input2, input3):
    del input2, input3
    anyspec = pl.BlockSpec(memory_space=pl.ANY)
    out0, out1 = pl.pallas_call(
        _copy_body,
        in_specs=[anyspec, anyspec],
        out_specs=[anyspec, anyspec],
        out_shape=[
            jax.ShapeDtypeStruct((M, B), jnp.float32),
            jax.ShapeDtypeStruct((M, B), jnp.float32),
        ],
        scratch_shapes=[
            pltpu.VMEM((SLOTS, CH, B), jnp.float32),
            pltpu.SemaphoreType.DMA((SLOTS,)),
            pltpu.SemaphoreType.DMA((SLOTS,)),
        ],
    )(input0, input1)
    return (out0, out1)


# speedup vs baseline: 1.0088x; 1.0088x over previous
import jax
import jax.numpy as jnp
from jax.experimental import pallas as pl
from jax.experimental.pallas import tpu as pltpu

M = 65536
B = 256
NCHUNK = 16
CH = M // NCHUNK
SLOTS = 6
LOOKAHEAD = 4


def _copy_body(i0_ref, i1_ref, o0_ref, o1_ref, buf, load_sems, store_sems):
    srcs = (i0_ref, i1_ref)
    dsts = (o0_ref, o1_ref)
    ntask = 2 * NCHUNK

    def load(t):
        a, c = t % 2, t // 2
        s = t % SLOTS
        pltpu.make_async_copy(
            srcs[a].at[pl.ds(c * CH, CH), :], buf.at[s], load_sems.at[s]
        ).start()

    def store(t):
        a, c = t % 2, t // 2
        s = t % SLOTS
        pltpu.make_async_copy(
            buf.at[s], dsts[a].at[pl.ds(c * CH, CH), :], store_sems.at[s]
        ).start()

    def wait_load(t):
        a, c = t % 2, t // 2
        s = t % SLOTS
        pltpu.make_async_copy(
            srcs[a].at[pl.ds(c * CH, CH), :], buf.at[s], load_sems.at[s]
        ).wait()

    def wait_store(t):
        a, c = t % 2, t // 2
        s = t % SLOTS
        pltpu.make_async_copy(
            buf.at[s], dsts[a].at[pl.ds(c * CH, CH), :], store_sems.at[s]
        ).wait()

    for t in range(LOOKAHEAD):
        load(t)
    for t in range(ntask):
        wait_load(t)
        store(t)
        u = t + LOOKAHEAD
        if u < ntask:
            if u >= SLOTS:
                wait_store(u - SLOTS)
            load(u)
    for t in range(ntask - SLOTS, ntask):
        wait_store(t)


def kernel(input0, input1, input2, input3):
    del input2, input3
    anyspec = pl.BlockSpec(memory_space=pl.ANY)
    out0, out1 = pl.pallas_call(
        _copy_body,
        in_specs=[anyspec, anyspec],
        out_specs=[anyspec, anyspec],
        out_shape=[
            jax.ShapeDtypeStruct((M, B), jnp.float32),
            jax.ShapeDtypeStruct((M, B), jnp.float32),
        ],
        scratch_shapes=[
            pltpu.VMEM((SLOTS, CH, B), jnp.float32),
            pltpu.SemaphoreType.DMA((SLOTS,)),
            pltpu.SemaphoreType.DMA((SLOTS,)),
        ],
    )(input0, input1)
    return (out0, out1)


# speedup vs baseline: 1.0122x; 1.0033x over previous
import jax
import jax.numpy as jnp
from jax.experimental import pallas as pl
from jax.experimental.pallas import tpu as pltpu

M = 65536
B = 256
NCHUNK = 8
CH = M // NCHUNK
SLOTS = 4
LOOKAHEAD = 3


def _copy_body(i0_ref, i1_ref, o0_ref, o1_ref, buf, load_sems, store_sems):
    srcs = (i0_ref, i1_ref)
    dsts = (o0_ref, o1_ref)
    ntask = 2 * NCHUNK

    def load(t):
        a, c = t % 2, t // 2
        s = t % SLOTS
        pltpu.make_async_copy(
            srcs[a].at[pl.ds(c * CH, CH), :], buf.at[s], load_sems.at[s]
        ).start()

    def store(t):
        a, c = t % 2, t // 2
        s = t % SLOTS
        pltpu.make_async_copy(
            buf.at[s], dsts[a].at[pl.ds(c * CH, CH), :], store_sems.at[s]
        ).start()

    def wait_load(t):
        a, c = t % 2, t // 2
        s = t % SLOTS
        pltpu.make_async_copy(
            srcs[a].at[pl.ds(c * CH, CH), :], buf.at[s], load_sems.at[s]
        ).wait()

    def wait_store(t):
        a, c = t % 2, t // 2
        s = t % SLOTS
        pltpu.make_async_copy(
            buf.at[s], dsts[a].at[pl.ds(c * CH, CH), :], store_sems.at[s]
        ).wait()

    for t in range(LOOKAHEAD):
        load(t)
    for t in range(ntask):
        wait_load(t)
        store(t)
        u = t + LOOKAHEAD
        if u < ntask:
            if u >= SLOTS:
                wait_store(u - SLOTS)
            load(u)
    for t in range(ntask - SLOTS, ntask):
        wait_store(t)


def kernel(input0, input1, input2, input3):
    del input2, input3
    anyspec = pl.BlockSpec(memory_space=pl.ANY)
    out0, out1 = pl.pallas_call(
        _copy_body,
        in_specs=[anyspec, anyspec],
        out_specs=[anyspec, anyspec],
        out_shape=[
            jax.ShapeDtypeStruct((M, B), jnp.float32),
            jax.ShapeDtypeStruct((M, B), jnp.float32),
        ],
        scratch_shapes=[
            pltpu.VMEM((SLOTS, CH, B), jnp.float32),
            pltpu.SemaphoreType.DMA((SLOTS,)),
            pltpu.SemaphoreType.DMA((SLOTS,)),
        ],
    )(input0, input1)
    return (out0, out1)


# speedup vs baseline: 1.0128x; 1.0006x over previous
import jax
import jax.numpy as jnp
from jax.experimental import pallas as pl
from jax.experimental.pallas import tpu as pltpu

M = 65536
B = 256
NCHUNK = 8
CH = M // NCHUNK
SLOTS = 6
LOOKAHEAD = 4


def _copy_body(i0_ref, i1_ref, o0_ref, o1_ref, buf, load_sems, store_sems):
    srcs = (i0_ref, i1_ref)
    dsts = (o0_ref, o1_ref)
    ntask = 2 * NCHUNK

    def load(t):
        a, c = t % 2, t // 2
        s = t % SLOTS
        pltpu.make_async_copy(
            srcs[a].at[pl.ds(c * CH, CH), :], buf.at[s], load_sems.at[s]
        ).start()

    def store(t):
        a, c = t % 2, t // 2
        s = t % SLOTS
        pltpu.make_async_copy(
            buf.at[s], dsts[a].at[pl.ds(c * CH, CH), :], store_sems.at[s]
        ).start()

    def wait_load(t):
        a, c = t % 2, t // 2
        s = t % SLOTS
        pltpu.make_async_copy(
            srcs[a].at[pl.ds(c * CH, CH), :], buf.at[s], load_sems.at[s]
        ).wait()

    def wait_store(t):
        a, c = t % 2, t // 2
        s = t % SLOTS
        pltpu.make_async_copy(
            buf.at[s], dsts[a].at[pl.ds(c * CH, CH), :], store_sems.at[s]
        ).wait()

    for t in range(LOOKAHEAD):
        load(t)
    for t in range(ntask):
        wait_load(t)
        store(t)
        u = t + LOOKAHEAD
        if u < ntask:
            if u >= SLOTS:
                wait_store(u - SLOTS)
            load(u)
    for t in range(ntask - SLOTS, ntask):
        wait_store(t)


def kernel(input0, input1, input2, input3):
    del input2, input3
    anyspec = pl.BlockSpec(memory_space=pl.ANY)
    out0, out1 = pl.pallas_call(
        _copy_body,
        in_specs=[anyspec, anyspec],
        out_specs=[anyspec, anyspec],
        out_shape=[
            jax.ShapeDtypeStruct((M, B), jnp.float32),
            jax.ShapeDtypeStruct((M, B), jnp.float32),
        ],
        scratch_shapes=[
            pltpu.VMEM((SLOTS, CH, B), jnp.float32),
            pltpu.SemaphoreType.DMA((SLOTS,)),
            pltpu.SemaphoreType.DMA((SLOTS,)),
        ],
    )(input0, input1)
    return (out0, out1)
